# Initial kernel scaffold; baseline (speedup 1.0000x reference)
import jax
import jax.numpy as jnp
from jax.experimental import pallas as pl


def kernel(probas, labels):
    raise NotImplementedError("write your pallas kernel here")


# speedup vs baseline: 92.3438x; 92.3438x over previous
import functools

import jax
import jax.numpy as jnp
from jax import lax
from jax.experimental import pallas as pl
from jax.experimental.pallas import tpu as pltpu
from jax.experimental.pallas import tpu_sc as plsc

_SHAPE = (1, 3, 64, 256, 256)
_P = _SHAPE[2] * _SHAPE[3] * _SHAPE[4]
_NW = 32
_TOK_PER_W = _P // _NW
_CH = 8192
_NCHUNK = _TOK_PER_W // _CH
_NV = _CH // 16


def _log_s(s):
    r = (s - 1.0) / (s + 1.0)
    r2 = r * r
    p = ((((r2 * (1.0 / 11.0) + 1.0 / 9.0) * r2 + 1.0 / 7.0) * r2
          + 1.0 / 5.0) * r2 + 1.0 / 3.0) * r2 + 1.0
    return (2.0 * r) * p


_mesh = plsc.VectorSubcoreMesh(core_axis_name="c", subcore_axis_name="s")


@functools.partial(
    pl.kernel,
    mesh=_mesh,
    out_type=jax.ShapeDtypeStruct((_NW, 96), jnp.float32),
    scratch_types=[
        pltpu.VMEM((_CH,), jnp.float32),
        pltpu.VMEM((_CH,), jnp.float32),
        pltpu.VMEM((_CH,), jnp.float32),
        pltpu.VMEM((_CH,), jnp.int32),
        pltpu.VMEM((_CH,), jnp.int32),
        pltpu.VMEM((96,), jnp.float32),
    ],
)
def _sc_partials(x_hbm, l_hbm, out_hbm, bx0, bx1, bx2, bl1, bl2, bout):
    wid = lax.axis_index("s") * 2 + lax.axis_index("c")
    base = wid * _TOK_PER_W

    def chunk_body(k, carry):
        off = base + k * _CH
        pltpu.sync_copy(x_hbm.at[pl.ds(off, _CH)], bx0)
        pltpu.sync_copy(x_hbm.at[pl.ds(_P + off, _CH)], bx1)
        pltpu.sync_copy(x_hbm.at[pl.ds(2 * _P + off, _CH)], bx2)
        pltpu.sync_copy(l_hbm.at[pl.ds(_P + off, _CH)], bl1)
        pltpu.sync_copy(l_hbm.at[pl.ds(2 * _P + off, _CH)], bl2)

        def vec_body(i, c):
            a0, a1, a2, n0, n1, n2 = c
            sl = pl.ds(i * 16, 16)
            x0 = bx0[sl]
            x1 = bx1[sl]
            x2 = bx2[sl]
            lab = bl1[sl] + 2 * bl2[sl]
            m = jnp.maximum(jnp.maximum(x0, x1), x2)
            s = jnp.exp(x0 - m) + jnp.exp(x1 - m) + jnp.exp(x2 - m)
            is0 = lab == 0
            is1 = lab == 1
            is2 = lab == 2
            x_sel = jnp.where(is0, x0, jnp.where(is1, x1, x2))
            err = _log_s(s) + (m - x_sel)
            zero = jnp.zeros_like(err)
            one = jnp.ones_like(err)
            return (a0 + jnp.where(is0, err, zero),
                    a1 + jnp.where(is1, err, zero),
                    a2 + jnp.where(is2, err, zero),
                    n0 + jnp.where(is0, one, zero),
                    n1 + jnp.where(is1, one, zero),
                    n2 + jnp.where(is2, one, zero))

        return lax.fori_loop(0, _NV, vec_body, carry)

    z = jnp.zeros((16,), jnp.float32)
    acc = lax.fori_loop(0, _NCHUNK, chunk_body, (z, z, z, z, z, z))
    for k in range(6):
        bout[pl.ds(k * 16, 16)] = acc[k]
    pltpu.sync_copy(bout, out_hbm.at[wid])


def kernel(probas, labels):
    x = probas.reshape(-1)
    l = labels.reshape(-1)
    parts = _sc_partials(x, l)
    p6 = parts.reshape(_NW, 6, 16).sum(axis=(0, 2))
    sums = p6[:3]
    counts = p6[3:]
    losses = jnp.where(counts > 0, sums / counts, 0.0)
    return jnp.mean(losses)
